# baseline (speedup 1.0000x reference)
import functools

import jax
import jax.numpy as jnp
from jax import lax
from jax.experimental import pallas as pl
from jax.experimental.pallas import tpu as pltpu
from jax.experimental.pallas import tpu_sc as plsc

_N_CAT = 26
_N_CONT = 13
_VOCAB_P1 = 100001
_CAT_DIM = 16
_CONT_DIM = 16

_NC = 2
_NS = 16
_NW = _NC * _NS


def _make_cat_gather(rows_total: int):
    rows_per_w = rows_total // _NW
    chunk = 1664
    while rows_per_w % chunk:
        chunk //= 2
    n_chunks = rows_per_w // chunk

    mesh = plsc.VectorSubcoreMesh(core_axis_name="c", subcore_axis_name="s")

    @functools.partial(
        pl.kernel,
        mesh=mesh,
        out_type=jax.ShapeDtypeStruct((rows_total, _CAT_DIM), jnp.float32),
        compiler_params=pltpu.CompilerParams(use_tc_tiling_on_sc=False),
        scratch_types=[
            pltpu.VMEM((chunk,), jnp.int32),
            pltpu.VMEM((chunk, _CAT_DIM), jnp.float32),
            pltpu.SemaphoreType.DMA,
        ],
    )
    def cat_gather(idx_hbm, table_hbm, out_hbm, idx_v, rows_v, sem):
        wid = lax.axis_index("s") * _NC + lax.axis_index("c")
        lane = lax.iota(jnp.int32, 16)

        def chunk_body(g, carry):
            base = wid * rows_per_w + g * chunk
            pltpu.sync_copy(idx_hbm.at[pl.ds(base, chunk)], idx_v)

            def off_body(t, c):
                p = base + t * 16
                off = ((lane + p) % _N_CAT) * _VOCAB_P1
                sl = pl.ds(t * 16, 16)
                idx_v[sl] = idx_v[sl] + off
                return c

            lax.fori_loop(0, chunk // 16, off_body, 0)
            pltpu.async_copy(table_hbm.at[idx_v], rows_v, sem).wait()
            pltpu.sync_copy(rows_v, out_hbm.at[pl.ds(base, chunk)])
            return carry

        lax.fori_loop(0, n_chunks, chunk_body, 0)

    return cat_gather


def _cont_body(x_ref, w_ref, o_ref):
    for s in range(_N_CONT):
        o_ref[:, s * _CONT_DIM:(s + 1) * _CONT_DIM] = (
            x_ref[:, s:s + 1] * w_ref[s:s + 1, :]
        )


def _cont_embed(x_cont, cont_w):
    b = x_cont.shape[0]
    nb = 2048
    grid = (b // nb,)
    return pl.pallas_call(
        _cont_body,
        grid=grid,
        in_specs=[
            pl.BlockSpec((nb, _N_CONT), lambda i: (i, 0)),
            pl.BlockSpec((_N_CONT, _CONT_DIM), lambda i: (0, 0)),
        ],
        out_specs=pl.BlockSpec((nb, _N_CONT * _CONT_DIM), lambda i: (i, 0)),
        out_shape=jax.ShapeDtypeStruct((b, _N_CONT * _CONT_DIM), jnp.float32),
    )(x_cont, cont_w)


def kernel(X, cat_tables, cont_w):
    b = X.shape[0]
    rows_total = b * _N_CAT
    idx_flat = X[:, :_N_CAT].astype(jnp.int32).reshape(rows_total)
    table2d = cat_tables.reshape(_N_CAT * _VOCAB_P1, _CAT_DIM)
    out2d = _make_cat_gather(rows_total)(idx_flat, table2d)
    x_cat = out2d.reshape(b, _N_CAT * _CAT_DIM)
    x_cont = _cont_embed(X[:, _N_CAT:_N_CAT + _N_CONT], cont_w)
    return (x_cat, x_cont)


# speedup vs baseline: 3.1312x; 3.1312x over previous
import functools

import jax
import jax.numpy as jnp
from jax import lax
from jax.experimental import pallas as pl
from jax.experimental.pallas import tpu as pltpu
from jax.experimental.pallas import tpu_sc as plsc

_N_CAT = 26
_N_CONT = 13
_VOCAB_P1 = 100001
_CAT_DIM = 16
_CONT_DIM = 16

_NC = 2
_NS = 16
_NW = _NC * _NS


def _make_cat_gather(b: int):
    chunk = b // _NW
    n_groups = chunk // 16

    mesh = plsc.VectorSubcoreMesh(core_axis_name="c", subcore_axis_name="s")

    @functools.partial(
        pl.kernel,
        mesh=mesh,
        out_type=jax.ShapeDtypeStruct((_N_CAT * _CAT_DIM, b), jnp.float32),
        scratch_types=(
            [pltpu.VMEM((chunk,), jnp.float32)]
            + [pltpu.VMEM((chunk,), jnp.int32) for _ in range(_CAT_DIM)]
            + [pltpu.VMEM((chunk,), jnp.float32) for _ in range(_CAT_DIM)]
            + [pltpu.SemaphoreType.DMA]
        ),
    )
    def cat_gather(xt_hbm, t1_hbm, out_hbm, idxf_v, *rest):
        idx_bufs = rest[:_CAT_DIM]
        val_bufs = rest[_CAT_DIM:2 * _CAT_DIM]
        sem = rest[2 * _CAT_DIM]
        wid = lax.axis_index("s") * _NC + lax.axis_index("c")
        b0 = wid * chunk

        def col_body(i, carry):
            pltpu.sync_copy(xt_hbm.at[i, pl.ds(b0, chunk)], idxf_v)
            base = i * _CAT_DIM * _VOCAB_P1

            def cvt_body(t, c):
                sl = pl.ds(t * 16, 16)
                raw = idxf_v[sl].astype(jnp.int32) + base
                for d in range(_CAT_DIM):
                    idx_bufs[d][sl] = raw + d * _VOCAB_P1
                return c

            lax.fori_loop(0, chunk // 16, cvt_body, 0)
            copies = []
            for d in range(_CAT_DIM):
                copies.append(
                    pltpu.async_copy(
                        t1_hbm.at[idx_bufs[d]], val_bufs[d], sem))
            for d in range(_CAT_DIM):
                copies[d].wait()
                pltpu.sync_copy(
                    val_bufs[d],
                    out_hbm.at[i * _CAT_DIM + d, pl.ds(b0, chunk)])
            return carry

        lax.fori_loop(0, _N_CAT, col_body, 0)

    return cat_gather


def _cont_body(x_ref, w_ref, o_ref):
    for s in range(_N_CONT):
        o_ref[s * _CONT_DIM:(s + 1) * _CONT_DIM, :] = (
            w_ref[s, :][:, None] * x_ref[_N_CAT + s, :][None, :]
        )


def _cont_embed_t(xt, cont_w):
    b = xt.shape[1]
    nb = 1024
    grid = (b // nb,)
    return pl.pallas_call(
        _cont_body,
        grid=grid,
        in_specs=[
            pl.BlockSpec((_N_CAT + _N_CONT, nb), lambda j: (0, j)),
            pl.BlockSpec((_N_CONT, _CONT_DIM), lambda j: (0, 0)),
        ],
        out_specs=pl.BlockSpec((_N_CONT * _CONT_DIM, nb), lambda j: (0, j)),
        out_shape=jax.ShapeDtypeStruct((_N_CONT * _CONT_DIM, b), jnp.float32),
    )(xt, cont_w)


def kernel(X, cat_tables, cont_w):
    b = X.shape[0]
    xt = X.T
    t1 = jnp.transpose(cat_tables, (0, 2, 1)).reshape(-1)
    out_cat_t = _make_cat_gather(b)(xt, t1)
    out_cont_t = _cont_embed_t(xt, cont_w)
    return (out_cat_t.T, out_cont_t.T)


# speedup vs baseline: 12.7124x; 4.0600x over previous
import functools

import jax
import jax.numpy as jnp
from jax import lax
from jax.experimental import pallas as pl
from jax.experimental.pallas import tpu as pltpu
from jax.experimental.pallas import tpu_sc as plsc

_N_CAT = 26
_N_CONT = 13
_VOCAB_P1 = 100001
_CAT_DIM = 16
_CONT_DIM = 16

_NC = 2
_NS = 16
_NW = _NC * _NS

_VPAD = 100096
_VMAIN = 99968
_TAILW = 40


def _make_detiler():
    n_tasks = _N_CAT * 2 * 50

    mesh = plsc.VectorSubcoreMesh(core_axis_name="c", subcore_axis_name="s")

    @functools.partial(
        pl.kernel,
        mesh=mesh,
        out_type=jax.ShapeDtypeStruct((_N_CAT * _CAT_DIM * _VPAD,),
                                      jnp.float32),
        compiler_params=pltpu.CompilerParams(
            use_tc_tiling_on_sc=True, needs_layout_passes=False),
        scratch_types=[
            pltpu.VMEM((8, 2048), jnp.float32),
            pltpu.VMEM((8, _TAILW), jnp.float32),
        ],
    )
    def detile(t3_hbm, tail_hbm, out_hbm, buf, tbuf):
        wid = lax.axis_index("s") * _NC + lax.axis_index("c")

        def task_body(k, carry):
            t = wid + k * _NW
            i = t // 100
            s = (t // 50) % 2
            j = t % 50
            r0 = pl.multiple_of(s * 8, 8)
            cbase = (i * _CAT_DIM + s * 8) * _VPAD

            @pl.when((t < n_tasks) & (j < 48))
            def _strip():
                v0 = pl.multiple_of(j * 2048, 128)
                pltpu.sync_copy(
                    t3_hbm.at[i, pl.ds(r0, 8), pl.ds(v0, 2048)], buf)
                for r in range(8):
                    pltpu.sync_copy(
                        buf.at[r],
                        out_hbm.at[pl.ds(
                            pl.multiple_of(cbase + r * _VPAD + v0, 128),
                            2048)])

            @pl.when((t < n_tasks) & (j == 48))
            def _strip_last():
                v0 = pl.multiple_of(98304, 128)
                pltpu.sync_copy(
                    t3_hbm.at[i, pl.ds(r0, 8), pl.ds(v0, 1664)],
                    buf.at[:, pl.ds(0, 1664)])
                for r in range(8):
                    pltpu.sync_copy(
                        buf.at[r, pl.ds(0, 1664)],
                        out_hbm.at[pl.ds(
                            pl.multiple_of(cbase + r * _VPAD + 98304, 128),
                            1664)])

            @pl.when((t < n_tasks) & (j == 49))
            def _tail():
                pltpu.sync_copy(tail_hbm.at[i, pl.ds(r0, 8), :], tbuf)
                for r in range(8):
                    pltpu.sync_copy(
                        tbuf.at[r],
                        out_hbm.at[pl.ds(
                            pl.multiple_of(cbase + r * _VPAD + _VMAIN, 8),
                            _TAILW)])

            return carry

        lax.fori_loop(0, pl.cdiv(n_tasks, _NW), task_body, 0)

    return detile


def _make_cat_gather(b: int):
    chunk = b // _NW
    n_groups = chunk // 16

    mesh = plsc.VectorSubcoreMesh(core_axis_name="c", subcore_axis_name="s")

    @functools.partial(
        pl.kernel,
        mesh=mesh,
        out_type=jax.ShapeDtypeStruct((_N_CAT * _CAT_DIM, b), jnp.float32),
        scratch_types=(
            [pltpu.VMEM((chunk,), jnp.float32)]
            + [pltpu.VMEM((chunk,), jnp.int32) for _ in range(_CAT_DIM)]
            + [pltpu.VMEM((chunk,), jnp.float32) for _ in range(_CAT_DIM)]
            + [pltpu.SemaphoreType.DMA]
        ),
    )
    def cat_gather(xt_hbm, t1_hbm, out_hbm, idxf_v, *rest):
        idx_bufs = rest[:_CAT_DIM]
        val_bufs = rest[_CAT_DIM:2 * _CAT_DIM]
        sem = rest[2 * _CAT_DIM]
        wid = lax.axis_index("s") * _NC + lax.axis_index("c")
        b0 = wid * chunk

        def col_body(i, carry):
            pltpu.sync_copy(xt_hbm.at[i, pl.ds(b0, chunk)], idxf_v)
            base = i * _CAT_DIM * _VPAD

            def cvt_body(t, c):
                sl = pl.ds(t * 16, 16)
                raw = idxf_v[sl].astype(jnp.int32) + base
                for d in range(_CAT_DIM):
                    idx_bufs[d][sl] = raw + d * _VPAD
                return c

            lax.fori_loop(0, chunk // 16, cvt_body, 0)
            copies = []
            for d in range(_CAT_DIM):
                copies.append(
                    pltpu.async_copy(
                        t1_hbm.at[idx_bufs[d]], val_bufs[d], sem))
            for d in range(_CAT_DIM):
                copies[d].wait()
                pltpu.sync_copy(
                    val_bufs[d],
                    out_hbm.at[i * _CAT_DIM + d, pl.ds(b0, chunk)])
            return carry

        lax.fori_loop(0, _N_CAT, col_body, 0)

    return cat_gather


def _cont_body(x_ref, w_ref, o_ref):
    for s in range(_N_CONT):
        o_ref[s * _CONT_DIM:(s + 1) * _CONT_DIM, :] = (
            w_ref[s, :][:, None] * x_ref[_N_CAT + s, :][None, :]
        )


def _cont_embed_t(xt, cont_w):
    b = xt.shape[1]
    nb = 1024
    grid = (b // nb,)
    return pl.pallas_call(
        _cont_body,
        grid=grid,
        in_specs=[
            pl.BlockSpec((_N_CAT + _N_CONT, nb), lambda j: (0, j)),
            pl.BlockSpec((_N_CONT, _CONT_DIM), lambda j: (0, 0)),
        ],
        out_specs=pl.BlockSpec((_N_CONT * _CONT_DIM, nb), lambda j: (0, j)),
        out_shape=jax.ShapeDtypeStruct((_N_CONT * _CONT_DIM, b), jnp.float32),
    )(xt, cont_w)


def kernel(X, cat_tables, cont_w):
    b = X.shape[0]
    xt = X.T
    t3 = jnp.transpose(cat_tables, (0, 2, 1))
    tail = jnp.pad(t3[:, :, _VMAIN:], ((0, 0), (0, 0),
                                       (0, _TAILW - (_VOCAB_P1 - _VMAIN))))
    t1 = _make_detiler()(t3, tail)
    out_cat_t = _make_cat_gather(b)(xt, t1)
    out_cont_t = _cont_embed_t(xt, cont_w)
    return (out_cat_t.T, out_cont_t.T)
